# Initial kernel scaffold; baseline (speedup 1.0000x reference)
import jax
import jax.numpy as jnp
from jax.experimental import pallas as pl


def kernel(x_enc, x_mark_enc, params):
    raise NotImplementedError("write your pallas kernel here")


# speedup vs baseline: 5.3369x; 5.3369x over previous
import math

import numpy as np
import jax
import jax.numpy as jnp
from jax.experimental import pallas as pl

B, L, ENC_IN = 4, 2048, 144
D_MODEL, N_HEADS, D_FF, E_LAYERS, FACTOR = 512, 8, 2048, 2, 5
D_HEAD = D_MODEL // N_HEADS
U = min(FACTOR * int(math.ceil(math.log(L))), L)
BL = B * L
_EPS = 1e-5

_index_sample = np.random.default_rng(0).integers(0, L, size=(L, U))
_COUNTS = np.zeros((L, L), np.float32)
np.add.at(_COUNTS, (np.repeat(np.arange(L), U), _index_sample.ravel()), 1.0)


def _pos_embedding():
    pos = np.arange(L)[:, None].astype(np.float32)
    div = np.exp(np.arange(0, D_MODEL, 2).astype(np.float32)
                 * -(math.log(10000.0) / D_MODEL))
    pe = np.zeros((L, D_MODEL), dtype=np.float32)
    pe[:, 0::2] = np.sin(pos * div)
    pe[:, 1::2] = np.cos(pos * div)
    return pe


_PE = _pos_embedding()


def _layer_norm(t, g, b):
    mu = jnp.mean(t, axis=1, keepdims=True)
    var = jnp.mean((t - mu) ** 2, axis=1, keepdims=True)
    return (t - mu) / jnp.sqrt(var + _EPS) * g + b


def _embed_body(x_ref, w_ref, pe_ref, o_ref):
    x = x_ref[...]
    w = w_ref[...]
    a0 = jnp.dot(x, w[:, :D_MODEL], preferred_element_type=jnp.float32)
    a1 = jnp.dot(x, w[:, D_MODEL:2 * D_MODEL], preferred_element_type=jnp.float32)
    a2 = jnp.dot(x, w[:, 2 * D_MODEL:], preferred_element_type=jnp.float32)
    out = (jnp.concatenate([a0[-1:], a0[:-1]], axis=0) + a1
           + jnp.concatenate([a2[1:], a2[:1]], axis=0) + pe_ref[...])
    o_ref[...] = out


def _qkv_body(x_ref, w_ref, b_ref, o_ref):
    o_ref[...] = (jnp.dot(x_ref[...], w_ref[...],
                          preferred_element_type=jnp.float32) + b_ref[...])


def _sample_body(q_ref, k_ref, c_ref, m_ref):
    c = c_ref[...]
    cmask = c > 0
    for h in range(N_HEADS):
        q = q_ref[:, h * D_HEAD:(h + 1) * D_HEAD]
        k = k_ref[:, h * D_HEAD:(h + 1) * D_HEAD]
        s = jax.lax.dot_general(q, k, (((1,), (1,)), ((), ())),
                                preferred_element_type=jnp.float32)
        mx = jnp.max(jnp.where(cmask, s, -jnp.inf), axis=1)
        sm = jnp.sum(s * c, axis=1) * (1.0 / L)
        m_ref[0, h, :] = mx - sm


def _topk_body(m_ref, o_ref):
    m = m_ref[...].reshape(B * N_HEADS, L)
    iota = jax.lax.broadcasted_iota(jnp.int32, (B * N_HEADS, L), 1)
    cols = []
    for _ in range(U):
        mx = jnp.max(m, axis=1, keepdims=True)
        cand = jnp.where(m == mx, iota, L)
        idx = jnp.min(cand, axis=1)
        cols.append(idx[:, None])
        m = jnp.where(iota == idx[:, None], -jnp.inf, m)
    o_ref[...] = jnp.concatenate(cols, axis=1).reshape(B, N_HEADS, U)


def _attn_body(mt_ref, q_ref, k_ref, v_ref, x_ref, wo_ref, bo_ref,
               g_ref, b_ref, o_ref):
    acc = x_ref[...] + bo_ref[...]
    for h in range(N_HEADS):
        idx = mt_ref[0, h, :]
        onehot = (idx[:, None] ==
                  jax.lax.broadcasted_iota(jnp.int32, (U, L), 1)
                  ).astype(jnp.float32)
        q = q_ref[:, h * D_HEAD:(h + 1) * D_HEAD]
        k = k_ref[:, h * D_HEAD:(h + 1) * D_HEAD]
        v = v_ref[:, h * D_HEAD:(h + 1) * D_HEAD]
        q_red = jnp.dot(onehot, q, preferred_element_type=jnp.float32)
        scores = jax.lax.dot_general(q_red, k, (((1,), (1,)), ((), ())),
                                     preferred_element_type=jnp.float32)
        scores = scores * (1.0 / math.sqrt(D_HEAD))
        mx = jnp.max(scores, axis=1, keepdims=True)
        e = jnp.exp(scores - mx)
        attn = e / jnp.sum(e, axis=1, keepdims=True)
        update = jnp.dot(attn, v, preferred_element_type=jnp.float32)
        vmean = jnp.mean(v, axis=0, keepdims=True)
        wo = wo_ref[h * D_HEAD:(h + 1) * D_HEAD, :]
        delta = jnp.dot(update - vmean, wo, preferred_element_type=jnp.float32)
        base = jnp.dot(vmean, wo, preferred_element_type=jnp.float32)
        onehot_t = (idx[None, :] ==
                    jax.lax.broadcasted_iota(jnp.int32, (L, U), 0)
                    ).astype(jnp.float32)
        acc = acc + jnp.dot(onehot_t, delta,
                            preferred_element_type=jnp.float32) + base
    o_ref[...] = _layer_norm(acc, g_ref[...], b_ref[...])


def _ffn_body(x_ref, w1_ref, b1_ref, w2_ref, b2_ref, g_ref, bb_ref, o_ref):
    x = x_ref[...]
    h1 = jnp.dot(x, w1_ref[...], preferred_element_type=jnp.float32) + b1_ref[...]
    h1 = 0.5 * h1 * (1.0 + jax.lax.erf(h1 * (1.0 / math.sqrt(2.0))))
    y = jnp.dot(h1, w2_ref[...], preferred_element_type=jnp.float32) + b2_ref[...]
    o_ref[...] = _layer_norm(x + y, g_ref[...], bb_ref[...])


def _final_body(x_ref, g_ref, b_ref, mark_ref, o_ref):
    o_ref[...] = _layer_norm(x_ref[...], g_ref[...], b_ref[...]) * mark_ref[...]


_F32 = jnp.float32


def _embed(x2, wcat, pe):
    return pl.pallas_call(
        _embed_body,
        grid=(B,),
        in_specs=[
            pl.BlockSpec((L, ENC_IN), lambda b: (b, 0)),
            pl.BlockSpec((ENC_IN, 3 * D_MODEL), lambda b: (0, 0)),
            pl.BlockSpec((L, D_MODEL), lambda b: (0, 0)),
        ],
        out_specs=pl.BlockSpec((L, D_MODEL), lambda b: (b, 0)),
        out_shape=jax.ShapeDtypeStruct((BL, D_MODEL), _F32),
    )(x2, wcat, pe)


def _qkv(x, wqkv, bqkv):
    blk = 512
    return pl.pallas_call(
        _qkv_body,
        grid=(BL // blk,),
        in_specs=[
            pl.BlockSpec((blk, D_MODEL), lambda i: (i, 0)),
            pl.BlockSpec((D_MODEL, 3 * D_MODEL), lambda i: (0, 0)),
            pl.BlockSpec((1, 3 * D_MODEL), lambda i: (0, 0)),
        ],
        out_specs=pl.BlockSpec((blk, 3 * D_MODEL), lambda i: (i, 0)),
        out_shape=jax.ShapeDtypeStruct((BL, 3 * D_MODEL), _F32),
    )(x, wqkv, bqkv)


_LB = 1024


def _sample(qkv, c):
    nlb = L // _LB
    return pl.pallas_call(
        _sample_body,
        grid=(nlb, B),
        in_specs=[
            pl.BlockSpec((_LB, D_MODEL), lambda lb, bb: (bb * nlb + lb, 0)),
            pl.BlockSpec((L, D_MODEL), lambda lb, bb: (bb, 1)),
            pl.BlockSpec((_LB, L), lambda lb, bb: (lb, 0)),
        ],
        out_specs=pl.BlockSpec((1, N_HEADS, _LB), lambda lb, bb: (bb, 0, lb)),
        out_shape=jax.ShapeDtypeStruct((B, N_HEADS, L), _F32),
    )(qkv, qkv, c)


def _topk(m):
    return pl.pallas_call(
        _topk_body,
        out_shape=jax.ShapeDtypeStruct((B, N_HEADS, U), jnp.int32),
    )(m)


def _attn(mt, qkv, x, wot, bo, g, b):
    return pl.pallas_call(
        _attn_body,
        grid=(B,),
        in_specs=[
            pl.BlockSpec((1, N_HEADS, U), lambda bb: (bb, 0, 0)),
            pl.BlockSpec((L, D_MODEL), lambda bb: (bb, 0)),
            pl.BlockSpec((L, D_MODEL), lambda bb: (bb, 1)),
            pl.BlockSpec((L, D_MODEL), lambda bb: (bb, 2)),
            pl.BlockSpec((L, D_MODEL), lambda bb: (bb, 0)),
            pl.BlockSpec((D_MODEL, D_MODEL), lambda bb: (0, 0)),
            pl.BlockSpec((1, D_MODEL), lambda bb: (0, 0)),
            pl.BlockSpec((1, D_MODEL), lambda bb: (0, 0)),
            pl.BlockSpec((1, D_MODEL), lambda bb: (0, 0)),
        ],
        out_specs=pl.BlockSpec((L, D_MODEL), lambda bb: (bb, 0)),
        out_shape=jax.ShapeDtypeStruct((BL, D_MODEL), _F32),
    )(mt, qkv, qkv, qkv, x, wot, bo, g, b)


def _ffn(x, w1, b1, w2, b2, g, bb):
    blk = 512
    return pl.pallas_call(
        _ffn_body,
        grid=(BL // blk,),
        in_specs=[
            pl.BlockSpec((blk, D_MODEL), lambda i: (i, 0)),
            pl.BlockSpec((D_MODEL, D_FF), lambda i: (0, 0)),
            pl.BlockSpec((1, D_FF), lambda i: (0, 0)),
            pl.BlockSpec((D_FF, D_MODEL), lambda i: (0, 0)),
            pl.BlockSpec((1, D_MODEL), lambda i: (0, 0)),
            pl.BlockSpec((1, D_MODEL), lambda i: (0, 0)),
            pl.BlockSpec((1, D_MODEL), lambda i: (0, 0)),
        ],
        out_specs=pl.BlockSpec((blk, D_MODEL), lambda i: (i, 0)),
        out_shape=jax.ShapeDtypeStruct((BL, D_MODEL), _F32),
    )(x, w1, b1, w2, b2, g, bb)


def _final(x, g, b, mark):
    blk = 512
    return pl.pallas_call(
        _final_body,
        grid=(BL // blk,),
        in_specs=[
            pl.BlockSpec((blk, D_MODEL), lambda i: (i, 0)),
            pl.BlockSpec((1, D_MODEL), lambda i: (0, 0)),
            pl.BlockSpec((1, D_MODEL), lambda i: (0, 0)),
            pl.BlockSpec((blk, 1), lambda i: (i, 0)),
        ],
        out_specs=pl.BlockSpec((blk, D_MODEL), lambda i: (i, 0)),
        out_shape=jax.ShapeDtypeStruct((BL, D_MODEL), _F32),
    )(x, g, b, mark)


def kernel(x_enc, x_mark_enc, params):
    c = jnp.asarray(_COUNTS)
    pe = jnp.asarray(_PE)
    w = params['token_conv_w']
    wcat = jnp.concatenate([w[:, :, 0].T, w[:, :, 1].T, w[:, :, 2].T], axis=1)
    x = _embed(x_enc.reshape(BL, ENC_IN), wcat, pe)
    for l in range(E_LAYERS):
        p = params['layer_%d' % l]
        wqkv = jnp.concatenate([p['wq'].T, p['wk'].T, p['wv'].T], axis=1)
        bqkv = jnp.concatenate([p['bq'], p['bk'], p['bv']])[None, :]
        qkv = _qkv(x, wqkv, bqkv)
        m = _sample(qkv, c)
        mt = _topk(m)
        x = _attn(mt, qkv, x, p['wo'].T, p['bo'][None], p['ln1_g'][None],
                  p['ln1_b'][None])
        x = _ffn(x, p['conv1_w'], p['conv1_b'][None], p['conv2_w'],
                 p['conv2_b'][None], p['ln2_g'][None], p['ln2_b'][None])
    out = _final(x, params['norm_g'][None], params['norm_b'][None],
                 x_mark_enc.reshape(BL, 1))
    return out.reshape(B, L * D_MODEL)


# speedup vs baseline: 5.6557x; 1.0597x over previous
import math

import numpy as np
import jax
import jax.numpy as jnp
from jax.experimental import pallas as pl

B, L, ENC_IN = 4, 2048, 144
D_MODEL, N_HEADS, D_FF, E_LAYERS, FACTOR = 512, 8, 2048, 2, 5
D_HEAD = D_MODEL // N_HEADS
U = min(FACTOR * int(math.ceil(math.log(L))), L)
BL = B * L
_EPS = 1e-5

_index_sample = np.random.default_rng(0).integers(0, L, size=(L, U))
_COUNTS = np.zeros((L, L), np.float32)
np.add.at(_COUNTS, (np.repeat(np.arange(L), U), _index_sample.ravel()), 1.0)


def _pos_embedding():
    pos = np.arange(L)[:, None].astype(np.float32)
    div = np.exp(np.arange(0, D_MODEL, 2).astype(np.float32)
                 * -(math.log(10000.0) / D_MODEL))
    pe = np.zeros((L, D_MODEL), dtype=np.float32)
    pe[:, 0::2] = np.sin(pos * div)
    pe[:, 1::2] = np.cos(pos * div)
    return pe


_PE = _pos_embedding()


def _layer_norm(t, g, b):
    mu = jnp.mean(t, axis=1, keepdims=True)
    var = jnp.mean((t - mu) ** 2, axis=1, keepdims=True)
    return (t - mu) / jnp.sqrt(var + _EPS) * g + b


def _embed_body(x_ref, w_ref, pe_ref, o_ref):
    x = x_ref[...].astype(jnp.bfloat16)
    w = w_ref[...]
    a0 = jnp.dot(x, w[:, :D_MODEL], preferred_element_type=jnp.float32)
    a1 = jnp.dot(x, w[:, D_MODEL:2 * D_MODEL], preferred_element_type=jnp.float32)
    a2 = jnp.dot(x, w[:, 2 * D_MODEL:], preferred_element_type=jnp.float32)
    out = (jnp.concatenate([a0[-1:], a0[:-1]], axis=0) + a1
           + jnp.concatenate([a2[1:], a2[:1]], axis=0) + pe_ref[...])
    o_ref[...] = out


def _qkv_body(x_ref, w_ref, b_ref, o_ref):
    o_ref[...] = (jnp.dot(x_ref[...].astype(jnp.bfloat16), w_ref[...],
                          preferred_element_type=jnp.float32)
                  + b_ref[...]).astype(jnp.bfloat16)


def _sample_body(q_ref, k_ref, c_ref, m_ref):
    c = c_ref[...].astype(jnp.float32)
    cmask = c > 0
    for h in range(N_HEADS):
        q = q_ref[:, h * D_HEAD:(h + 1) * D_HEAD]
        k = k_ref[:, h * D_HEAD:(h + 1) * D_HEAD]
        s = jax.lax.dot_general(q, k, (((1,), (1,)), ((), ())),
                                preferred_element_type=jnp.float32)
        mx = jnp.max(jnp.where(cmask, s, -jnp.inf), axis=1)
        sm = jnp.sum(s * c, axis=1) * (1.0 / L)
        m_ref[0, h, :] = mx - sm


def _topk_body(m_ref, o_ref):
    m = m_ref[...].reshape(B * N_HEADS, L)
    iota = jax.lax.broadcasted_iota(jnp.int32, (B * N_HEADS, L), 1)
    cols = []
    for _ in range(U):
        mx = jnp.max(m, axis=1, keepdims=True)
        cand = jnp.where(m == mx, iota, L)
        idx = jnp.min(cand, axis=1)
        cols.append(idx[:, None])
        m = jnp.where(iota == idx[:, None], -jnp.inf, m)
    o_ref[...] = jnp.concatenate(cols, axis=1).reshape(B, N_HEADS, U)


def _attn_body(mt_ref, q_ref, k_ref, v_ref, x_ref, wo_ref, bo_ref,
               g_ref, b_ref, o_ref):
    acc = x_ref[...] + bo_ref[...]
    for h in range(N_HEADS):
        idx = mt_ref[0, h, :]
        onehot = (idx[:, None] ==
                  jax.lax.broadcasted_iota(jnp.int32, (U, L), 1)
                  ).astype(jnp.bfloat16)
        q = q_ref[:, h * D_HEAD:(h + 1) * D_HEAD]
        k = k_ref[:, h * D_HEAD:(h + 1) * D_HEAD]
        v = v_ref[:, h * D_HEAD:(h + 1) * D_HEAD]
        q_red = jnp.dot(onehot, q, preferred_element_type=jnp.float32)
        scores = jax.lax.dot_general(q_red.astype(jnp.bfloat16), k,
                                     (((1,), (1,)), ((), ())),
                                     preferred_element_type=jnp.float32)
        scores = scores * (1.0 / math.sqrt(D_HEAD))
        mx = jnp.max(scores, axis=1, keepdims=True)
        e = jnp.exp(scores - mx)
        attn = e / jnp.sum(e, axis=1, keepdims=True)
        update = jnp.dot(attn.astype(jnp.bfloat16), v,
                         preferred_element_type=jnp.float32)
        vmean = jnp.mean(v.astype(jnp.float32), axis=0, keepdims=True)
        wo = wo_ref[h * D_HEAD:(h + 1) * D_HEAD, :]
        delta = jnp.dot((update - vmean).astype(jnp.bfloat16), wo,
                        preferred_element_type=jnp.float32)
        base = jnp.dot(vmean.astype(jnp.bfloat16), wo,
                       preferred_element_type=jnp.float32)
        onehot_t = (idx[None, :] ==
                    jax.lax.broadcasted_iota(jnp.int32, (L, U), 0)
                    ).astype(jnp.bfloat16)
        acc = acc + jnp.dot(onehot_t, delta.astype(jnp.bfloat16),
                            preferred_element_type=jnp.float32) + base
    o_ref[...] = _layer_norm(acc, g_ref[...], b_ref[...])


def _ffn_body(x_ref, w1_ref, b1_ref, w2_ref, b2_ref, g_ref, bb_ref, o_ref):
    x = x_ref[...]
    h1 = jnp.dot(x.astype(jnp.bfloat16), w1_ref[...],
                 preferred_element_type=jnp.float32) + b1_ref[...]
    h1 = 0.5 * h1 * (1.0 + jax.lax.erf(h1 * (1.0 / math.sqrt(2.0))))
    y = jnp.dot(h1.astype(jnp.bfloat16), w2_ref[...],
                preferred_element_type=jnp.float32) + b2_ref[...]
    o_ref[...] = _layer_norm(x + y, g_ref[...], bb_ref[...])


def _final_body(x_ref, g_ref, b_ref, mark_ref, o_ref):
    o_ref[...] = _layer_norm(x_ref[...], g_ref[...], b_ref[...]) * mark_ref[...]


_F32 = jnp.float32


def _embed(x2, wcat, pe):
    return pl.pallas_call(
        _embed_body,
        grid=(B,),
        in_specs=[
            pl.BlockSpec((L, ENC_IN), lambda b: (b, 0)),
            pl.BlockSpec((ENC_IN, 3 * D_MODEL), lambda b: (0, 0)),
            pl.BlockSpec((L, D_MODEL), lambda b: (0, 0)),
        ],
        out_specs=pl.BlockSpec((L, D_MODEL), lambda b: (b, 0)),
        out_shape=jax.ShapeDtypeStruct((BL, D_MODEL), _F32),
    )(x2, wcat, pe)


def _qkv(x, wqkv, bqkv):
    blk = 512
    return pl.pallas_call(
        _qkv_body,
        grid=(BL // blk,),
        in_specs=[
            pl.BlockSpec((blk, D_MODEL), lambda i: (i, 0)),
            pl.BlockSpec((D_MODEL, 3 * D_MODEL), lambda i: (0, 0)),
            pl.BlockSpec((1, 3 * D_MODEL), lambda i: (0, 0)),
        ],
        out_specs=pl.BlockSpec((blk, 3 * D_MODEL), lambda i: (i, 0)),
        out_shape=jax.ShapeDtypeStruct((BL, 3 * D_MODEL), jnp.bfloat16),
    )(x, wqkv, bqkv)


_LB = 1024


def _sample(qkv, c):
    nlb = L // _LB
    return pl.pallas_call(
        _sample_body,
        grid=(nlb, B),
        in_specs=[
            pl.BlockSpec((_LB, D_MODEL), lambda lb, bb: (bb * nlb + lb, 0)),
            pl.BlockSpec((L, D_MODEL), lambda lb, bb: (bb, 1)),
            pl.BlockSpec((_LB, L), lambda lb, bb: (lb, 0)),
        ],
        out_specs=pl.BlockSpec((1, N_HEADS, _LB), lambda lb, bb: (bb, 0, lb)),
        out_shape=jax.ShapeDtypeStruct((B, N_HEADS, L), _F32),
    )(qkv, qkv, c)


def _topk(m):
    return pl.pallas_call(
        _topk_body,
        out_shape=jax.ShapeDtypeStruct((B, N_HEADS, U), jnp.int32),
    )(m)


def _attn(mt, qkv, x, wot, bo, g, b):
    return pl.pallas_call(
        _attn_body,
        grid=(B,),
        in_specs=[
            pl.BlockSpec((1, N_HEADS, U), lambda bb: (bb, 0, 0)),
            pl.BlockSpec((L, D_MODEL), lambda bb: (bb, 0)),
            pl.BlockSpec((L, D_MODEL), lambda bb: (bb, 1)),
            pl.BlockSpec((L, D_MODEL), lambda bb: (bb, 2)),
            pl.BlockSpec((L, D_MODEL), lambda bb: (bb, 0)),
            pl.BlockSpec((D_MODEL, D_MODEL), lambda bb: (0, 0)),
            pl.BlockSpec((1, D_MODEL), lambda bb: (0, 0)),
            pl.BlockSpec((1, D_MODEL), lambda bb: (0, 0)),
            pl.BlockSpec((1, D_MODEL), lambda bb: (0, 0)),
        ],
        out_specs=pl.BlockSpec((L, D_MODEL), lambda bb: (bb, 0)),
        out_shape=jax.ShapeDtypeStruct((BL, D_MODEL), _F32),
    )(mt, qkv, qkv, qkv, x, wot, bo, g, b)


def _ffn(x, w1, b1, w2, b2, g, bb):
    blk = 512
    return pl.pallas_call(
        _ffn_body,
        grid=(BL // blk,),
        in_specs=[
            pl.BlockSpec((blk, D_MODEL), lambda i: (i, 0)),
            pl.BlockSpec((D_MODEL, D_FF), lambda i: (0, 0)),
            pl.BlockSpec((1, D_FF), lambda i: (0, 0)),
            pl.BlockSpec((D_FF, D_MODEL), lambda i: (0, 0)),
            pl.BlockSpec((1, D_MODEL), lambda i: (0, 0)),
            pl.BlockSpec((1, D_MODEL), lambda i: (0, 0)),
            pl.BlockSpec((1, D_MODEL), lambda i: (0, 0)),
        ],
        out_specs=pl.BlockSpec((blk, D_MODEL), lambda i: (i, 0)),
        out_shape=jax.ShapeDtypeStruct((BL, D_MODEL), _F32),
    )(x, w1, b1, w2, b2, g, bb)


def _final(x, g, b, mark):
    blk = 512
    return pl.pallas_call(
        _final_body,
        grid=(BL // blk,),
        in_specs=[
            pl.BlockSpec((blk, D_MODEL), lambda i: (i, 0)),
            pl.BlockSpec((1, D_MODEL), lambda i: (0, 0)),
            pl.BlockSpec((1, D_MODEL), lambda i: (0, 0)),
            pl.BlockSpec((blk, 1), lambda i: (i, 0)),
        ],
        out_specs=pl.BlockSpec((blk, D_MODEL), lambda i: (i, 0)),
        out_shape=jax.ShapeDtypeStruct((BL, D_MODEL), _F32),
    )(x, g, b, mark)


def kernel(x_enc, x_mark_enc, params):
    c = jnp.asarray(_COUNTS).astype(jnp.bfloat16)
    pe = jnp.asarray(_PE)
    w = params['token_conv_w']
    wcat = jnp.concatenate([w[:, :, 0].T, w[:, :, 1].T, w[:, :, 2].T],
                           axis=1).astype(jnp.bfloat16)
    x = _embed(x_enc.reshape(BL, ENC_IN), wcat, pe)
    for l in range(E_LAYERS):
        p = params['layer_%d' % l]
        wqkv = jnp.concatenate([p['wq'].T, p['wk'].T, p['wv'].T],
                               axis=1).astype(jnp.bfloat16)
        bqkv = jnp.concatenate([p['bq'], p['bk'], p['bv']])[None, :]
        qkv = _qkv(x, wqkv, bqkv)
        m = _sample(qkv, c)
        mt = _topk(m)
        x = _attn(mt, qkv, x, p['wo'].T.astype(jnp.bfloat16), p['bo'][None],
                  p['ln1_g'][None], p['ln1_b'][None])
        x = _ffn(x, p['conv1_w'].astype(jnp.bfloat16), p['conv1_b'][None],
                 p['conv2_w'].astype(jnp.bfloat16), p['conv2_b'][None],
                 p['ln2_g'][None], p['ln2_b'][None])
    out = _final(x, params['norm_g'][None], params['norm_b'][None],
                 x_mark_enc.reshape(BL, 1))
    return out.reshape(B, L * D_MODEL)


# speedup vs baseline: 5.6965x; 1.0072x over previous
import math

import numpy as np
import jax
import jax.numpy as jnp
from jax.experimental import pallas as pl

B, L, ENC_IN = 4, 2048, 144
D_MODEL, N_HEADS, D_FF, E_LAYERS, FACTOR = 512, 8, 2048, 2, 5
D_HEAD = D_MODEL // N_HEADS
U = min(FACTOR * int(math.ceil(math.log(L))), L)
BL = B * L
_EPS = 1e-5

_index_sample = np.random.default_rng(0).integers(0, L, size=(L, U))
_COUNTS = np.zeros((L, L), np.float32)
np.add.at(_COUNTS, (np.repeat(np.arange(L), U), _index_sample.ravel()), 1.0)
_SBIAS = np.where(_COUNTS > 0, 0.0, -1e30).astype(np.float32)


def _pos_embedding():
    pos = np.arange(L)[:, None].astype(np.float32)
    div = np.exp(np.arange(0, D_MODEL, 2).astype(np.float32)
                 * -(math.log(10000.0) / D_MODEL))
    pe = np.zeros((L, D_MODEL), dtype=np.float32)
    pe[:, 0::2] = np.sin(pos * div)
    pe[:, 1::2] = np.cos(pos * div)
    return pe


_PE = _pos_embedding()


def _layer_norm(t, g, b):
    mu = jnp.mean(t, axis=1, keepdims=True)
    var = jnp.mean((t - mu) ** 2, axis=1, keepdims=True)
    return (t - mu) / jnp.sqrt(var + _EPS) * g + b


def _embed_body(x_ref, w_ref, pe_ref, o_ref):
    x = x_ref[...].astype(jnp.bfloat16)
    w = w_ref[...]
    a0 = jnp.dot(x, w[:, :D_MODEL], preferred_element_type=jnp.float32)
    a1 = jnp.dot(x, w[:, D_MODEL:2 * D_MODEL], preferred_element_type=jnp.float32)
    a2 = jnp.dot(x, w[:, 2 * D_MODEL:], preferred_element_type=jnp.float32)
    out = (jnp.concatenate([a0[-1:], a0[:-1]], axis=0) + a1
           + jnp.concatenate([a2[1:], a2[:1]], axis=0) + pe_ref[...])
    o_ref[...] = out


def _qkv_body(x_ref, w_ref, b_ref, o_ref):
    o_ref[...] = (jnp.dot(x_ref[...].astype(jnp.bfloat16), w_ref[...],
                          preferred_element_type=jnp.float32)
                  + b_ref[...]).astype(jnp.bfloat16)


def _sample_body(q_ref, k_ref, c_ref, bias_ref, m_ref):
    ck = jnp.dot(c_ref[...], k_ref[...],
                 preferred_element_type=jnp.float32)
    bias = bias_ref[...]
    for h in range(N_HEADS):
        q = q_ref[:, h * D_HEAD:(h + 1) * D_HEAD]
        k = k_ref[:, h * D_HEAD:(h + 1) * D_HEAD]
        s = jax.lax.dot_general(q, k, (((1,), (1,)), ((), ())),
                                preferred_element_type=jnp.float32)
        mx = jnp.max(s + bias, axis=1)
        sm = jnp.sum(q.astype(jnp.float32)
                     * ck[:, h * D_HEAD:(h + 1) * D_HEAD], axis=1) * (1.0 / L)
        m_ref[0, h, :] = mx - sm


def _topk_body(m_ref, o_ref):
    m = m_ref[...].reshape(B * N_HEADS, L)
    iota = jax.lax.broadcasted_iota(jnp.int32, (B * N_HEADS, L), 1)
    cols = []
    for _ in range(U):
        mx = jnp.max(m, axis=1, keepdims=True)
        cand = jnp.where(m == mx, iota, L)
        idx = jnp.min(cand, axis=1)
        cols.append(idx[:, None])
        m = jnp.where(iota == idx[:, None], -jnp.inf, m)
    o_ref[...] = jnp.concatenate(cols, axis=1).reshape(B, N_HEADS, U)


def _attn_body(mt_ref, q_ref, k_ref, v_ref, x_ref, wo_ref, bo_ref,
               g_ref, b_ref, o_ref):
    vmean_full = jnp.mean(v_ref[...].astype(jnp.float32), axis=0,
                          keepdims=True)
    base = jnp.dot(vmean_full.astype(jnp.bfloat16), wo_ref[...],
                   preferred_element_type=jnp.float32)
    iota_ul = jax.lax.broadcasted_iota(jnp.int32, (U, L), 1)
    iota_lu = jax.lax.broadcasted_iota(jnp.int32, (L, U), 0)
    deltas, ohts = [], []
    for h in range(N_HEADS):
        idx = mt_ref[0, h, :]
        onehot = (idx[:, None] == iota_ul).astype(jnp.bfloat16)
        q = q_ref[:, h * D_HEAD:(h + 1) * D_HEAD]
        k = k_ref[:, h * D_HEAD:(h + 1) * D_HEAD]
        v = v_ref[:, h * D_HEAD:(h + 1) * D_HEAD]
        q_red = jnp.dot(onehot, q, preferred_element_type=jnp.float32)
        scores = jax.lax.dot_general(q_red.astype(jnp.bfloat16), k,
                                     (((1,), (1,)), ((), ())),
                                     preferred_element_type=jnp.float32)
        scores = scores * (1.0 / math.sqrt(D_HEAD))
        mx = jnp.max(scores, axis=1, keepdims=True)
        e = jnp.exp(scores - mx)
        attn = e / jnp.sum(e, axis=1, keepdims=True)
        update = jnp.dot(attn.astype(jnp.bfloat16), v,
                         preferred_element_type=jnp.float32)
        diff = update - vmean_full[:, h * D_HEAD:(h + 1) * D_HEAD]
        wo = wo_ref[h * D_HEAD:(h + 1) * D_HEAD, :]
        deltas.append(jnp.dot(diff.astype(jnp.bfloat16), wo,
                              preferred_element_type=jnp.float32))
        ohts.append((idx[None, :] == iota_lu).astype(jnp.bfloat16))
    delta_all = jnp.concatenate(deltas, axis=0).astype(jnp.bfloat16)
    oht_all = jnp.concatenate(ohts, axis=1)
    acc = (x_ref[...] + bo_ref[...] + base
           + jnp.dot(oht_all, delta_all, preferred_element_type=jnp.float32))
    o_ref[...] = _layer_norm(acc, g_ref[...], b_ref[...])


def _ffn_body(x_ref, w1_ref, b1_ref, w2_ref, b2_ref, g_ref, bb_ref, o_ref):
    x = x_ref[...]
    h1 = jnp.dot(x.astype(jnp.bfloat16), w1_ref[...],
                 preferred_element_type=jnp.float32) + b1_ref[...]
    h1 = 0.5 * h1 * (1.0 + jax.lax.erf(h1 * (1.0 / math.sqrt(2.0))))
    y = jnp.dot(h1.astype(jnp.bfloat16), w2_ref[...],
                preferred_element_type=jnp.float32) + b2_ref[...]
    o_ref[...] = _layer_norm(x + y, g_ref[...], bb_ref[...])


def _final_body(x_ref, g_ref, b_ref, mark_ref, o_ref):
    o_ref[...] = _layer_norm(x_ref[...], g_ref[...], b_ref[...]) * mark_ref[...]


_F32 = jnp.float32


def _embed(x2, wcat, pe):
    return pl.pallas_call(
        _embed_body,
        grid=(B,),
        in_specs=[
            pl.BlockSpec((L, ENC_IN), lambda b: (b, 0)),
            pl.BlockSpec((ENC_IN, 3 * D_MODEL), lambda b: (0, 0)),
            pl.BlockSpec((L, D_MODEL), lambda b: (0, 0)),
        ],
        out_specs=pl.BlockSpec((L, D_MODEL), lambda b: (b, 0)),
        out_shape=jax.ShapeDtypeStruct((BL, D_MODEL), _F32),
    )(x2, wcat, pe)


def _qkv(x, wqkv, bqkv):
    blk = 512
    return pl.pallas_call(
        _qkv_body,
        grid=(BL // blk,),
        in_specs=[
            pl.BlockSpec((blk, D_MODEL), lambda i: (i, 0)),
            pl.BlockSpec((D_MODEL, 3 * D_MODEL), lambda i: (0, 0)),
            pl.BlockSpec((1, 3 * D_MODEL), lambda i: (0, 0)),
        ],
        out_specs=pl.BlockSpec((blk, 3 * D_MODEL), lambda i: (i, 0)),
        out_shape=jax.ShapeDtypeStruct((BL, 3 * D_MODEL), jnp.bfloat16),
    )(x, wqkv, bqkv)


_LB = 1024


def _sample(qkv, c, bias):
    nlb = L // _LB
    return pl.pallas_call(
        _sample_body,
        grid=(nlb, B),
        in_specs=[
            pl.BlockSpec((_LB, D_MODEL), lambda lb, bb: (bb * nlb + lb, 0)),
            pl.BlockSpec((L, D_MODEL), lambda lb, bb: (bb, 1)),
            pl.BlockSpec((_LB, L), lambda lb, bb: (lb, 0)),
            pl.BlockSpec((_LB, L), lambda lb, bb: (lb, 0)),
        ],
        out_specs=pl.BlockSpec((1, N_HEADS, _LB), lambda lb, bb: (bb, 0, lb)),
        out_shape=jax.ShapeDtypeStruct((B, N_HEADS, L), _F32),
    )(qkv, qkv, c, bias)


def _topk(m):
    return pl.pallas_call(
        _topk_body,
        out_shape=jax.ShapeDtypeStruct((B, N_HEADS, U), jnp.int32),
    )(m)


def _attn(mt, qkv, x, wot, bo, g, b):
    return pl.pallas_call(
        _attn_body,
        grid=(B,),
        in_specs=[
            pl.BlockSpec((1, N_HEADS, U), lambda bb: (bb, 0, 0)),
            pl.BlockSpec((L, D_MODEL), lambda bb: (bb, 0)),
            pl.BlockSpec((L, D_MODEL), lambda bb: (bb, 1)),
            pl.BlockSpec((L, D_MODEL), lambda bb: (bb, 2)),
            pl.BlockSpec((L, D_MODEL), lambda bb: (bb, 0)),
            pl.BlockSpec((D_MODEL, D_MODEL), lambda bb: (0, 0)),
            pl.BlockSpec((1, D_MODEL), lambda bb: (0, 0)),
            pl.BlockSpec((1, D_MODEL), lambda bb: (0, 0)),
            pl.BlockSpec((1, D_MODEL), lambda bb: (0, 0)),
        ],
        out_specs=pl.BlockSpec((L, D_MODEL), lambda bb: (bb, 0)),
        out_shape=jax.ShapeDtypeStruct((BL, D_MODEL), _F32),
    )(mt, qkv, qkv, qkv, x, wot, bo, g, b)


def _ffn(x, w1, b1, w2, b2, g, bb):
    blk = 512
    return pl.pallas_call(
        _ffn_body,
        grid=(BL // blk,),
        in_specs=[
            pl.BlockSpec((blk, D_MODEL), lambda i: (i, 0)),
            pl.BlockSpec((D_MODEL, D_FF), lambda i: (0, 0)),
            pl.BlockSpec((1, D_FF), lambda i: (0, 0)),
            pl.BlockSpec((D_FF, D_MODEL), lambda i: (0, 0)),
            pl.BlockSpec((1, D_MODEL), lambda i: (0, 0)),
            pl.BlockSpec((1, D_MODEL), lambda i: (0, 0)),
            pl.BlockSpec((1, D_MODEL), lambda i: (0, 0)),
        ],
        out_specs=pl.BlockSpec((blk, D_MODEL), lambda i: (i, 0)),
        out_shape=jax.ShapeDtypeStruct((BL, D_MODEL), _F32),
    )(x, w1, b1, w2, b2, g, bb)


def _final(x, g, b, mark):
    blk = 512
    return pl.pallas_call(
        _final_body,
        grid=(BL // blk,),
        in_specs=[
            pl.BlockSpec((blk, D_MODEL), lambda i: (i, 0)),
            pl.BlockSpec((1, D_MODEL), lambda i: (0, 0)),
            pl.BlockSpec((1, D_MODEL), lambda i: (0, 0)),
            pl.BlockSpec((blk, 1), lambda i: (i, 0)),
        ],
        out_specs=pl.BlockSpec((blk, D_MODEL), lambda i: (i, 0)),
        out_shape=jax.ShapeDtypeStruct((BL, D_MODEL), _F32),
    )(x, g, b, mark)


def kernel(x_enc, x_mark_enc, params):
    c = jnp.asarray(_COUNTS).astype(jnp.bfloat16)
    sbias = jnp.asarray(_SBIAS)
    pe = jnp.asarray(_PE)
    w = params['token_conv_w']
    wcat = jnp.concatenate([w[:, :, 0].T, w[:, :, 1].T, w[:, :, 2].T],
                           axis=1).astype(jnp.bfloat16)
    x = _embed(x_enc.reshape(BL, ENC_IN), wcat, pe)
    for l in range(E_LAYERS):
        p = params['layer_%d' % l]
        wqkv = jnp.concatenate([p['wq'].T, p['wk'].T, p['wv'].T],
                               axis=1).astype(jnp.bfloat16)
        bqkv = jnp.concatenate([p['bq'], p['bk'], p['bv']])[None, :]
        qkv = _qkv(x, wqkv, bqkv)
        m = _sample(qkv, c, sbias)
        mt = _topk(m)
        x = _attn(mt, qkv, x, p['wo'].T.astype(jnp.bfloat16), p['bo'][None],
                  p['ln1_g'][None], p['ln1_b'][None])
        x = _ffn(x, p['conv1_w'].astype(jnp.bfloat16), p['conv1_b'][None],
                 p['conv2_w'].astype(jnp.bfloat16), p['conv2_b'][None],
                 p['ln2_g'][None], p['ln2_b'][None])
    out = _final(x, params['norm_g'][None], params['norm_b'][None],
                 x_mark_enc.reshape(BL, 1))
    return out.reshape(B, L * D_MODEL)


# speedup vs baseline: 6.8642x; 1.2050x over previous
import math

import numpy as np
import jax
import jax.numpy as jnp
from jax.experimental import pallas as pl

B, L, ENC_IN = 4, 2048, 144
D_MODEL, N_HEADS, D_FF, E_LAYERS, FACTOR = 512, 8, 2048, 2, 5
D_HEAD = D_MODEL // N_HEADS
U = min(FACTOR * int(math.ceil(math.log(L))), L)
BL = B * L
_EPS = 1e-5

_index_sample = np.random.default_rng(0).integers(0, L, size=(L, U))
_COUNTS = np.zeros((L, L), np.float32)
np.add.at(_COUNTS, (np.repeat(np.arange(L), U), _index_sample.ravel()), 1.0)


def _pos_embedding():
    pos = np.arange(L)[:, None].astype(np.float32)
    div = np.exp(np.arange(0, D_MODEL, 2).astype(np.float32)
                 * -(math.log(10000.0) / D_MODEL))
    pe = np.zeros((L, D_MODEL), dtype=np.float32)
    pe[:, 0::2] = np.sin(pos * div)
    pe[:, 1::2] = np.cos(pos * div)
    return pe


_PE = _pos_embedding()


def _layer_norm(t, g, b):
    mu = jnp.mean(t, axis=1, keepdims=True)
    var = jnp.mean((t - mu) ** 2, axis=1, keepdims=True)
    return (t - mu) / jnp.sqrt(var + _EPS) * g + b


def _embed_body(x_ref, w_ref, pe_ref, o_ref):
    x = x_ref[...].astype(jnp.bfloat16)
    w = w_ref[...]
    a0 = jnp.dot(x, w[:, :D_MODEL], preferred_element_type=jnp.float32)
    a1 = jnp.dot(x, w[:, D_MODEL:2 * D_MODEL], preferred_element_type=jnp.float32)
    a2 = jnp.dot(x, w[:, 2 * D_MODEL:], preferred_element_type=jnp.float32)
    out = (jnp.concatenate([a0[-1:], a0[:-1]], axis=0) + a1
           + jnp.concatenate([a2[1:], a2[:1]], axis=0) + pe_ref[...])
    o_ref[...] = out


def _qkv_body(x_ref, w_ref, b_ref, o_ref):
    o_ref[...] = (jnp.dot(x_ref[...].astype(jnp.bfloat16), w_ref[...],
                          preferred_element_type=jnp.float32)
                  + b_ref[...]).astype(jnp.bfloat16)


def _sample_body(q_ref, k_ref, c_ref, m_ref):
    ck = jnp.dot(c_ref[...], k_ref[...],
                 preferred_element_type=jnp.float32)
    mask = c_ref[...] > 0
    cols = []
    for h in range(N_HEADS):
        q = q_ref[:, h * D_HEAD:(h + 1) * D_HEAD]
        k = k_ref[:, h * D_HEAD:(h + 1) * D_HEAD]
        s = jax.lax.dot_general(q, k, (((1,), (1,)), ((), ())),
                                preferred_element_type=jnp.float32)
        mx = jnp.max(jnp.where(mask, s, -1e30), axis=1, keepdims=True)
        sm = jnp.sum(q.astype(jnp.float32)
                     * ck[:, h * D_HEAD:(h + 1) * D_HEAD],
                     axis=1, keepdims=True) * (1.0 / L)
        cols.append(mx - sm)
    m_ref[...] = jnp.concatenate(cols, axis=1)


def _topk_body(m_ref, o_ref):
    m = m_ref[...].reshape(B, L, N_HEADS).transpose(0, 2, 1).reshape(
        B * N_HEADS, L)
    iota = jax.lax.broadcasted_iota(jnp.int32, (B * N_HEADS, L), 1)
    cols = []
    for _ in range(U):
        mx = jnp.max(m, axis=1, keepdims=True)
        cand = jnp.where(m == mx, iota, L)
        idx = jnp.min(cand, axis=1)
        cols.append(idx[:, None])
        m = jnp.where(iota == idx[:, None], -jnp.inf, m)
    o_ref[...] = jnp.concatenate(cols, axis=1).reshape(B, N_HEADS, U)


def _attn_body(mt_ref, q_ref, k_ref, v_ref, x_ref, wo_ref, bo_ref,
               g_ref, b_ref, o_ref):
    vmean_full = jnp.mean(v_ref[...].astype(jnp.float32), axis=0,
                          keepdims=True)
    base = jnp.dot(vmean_full.astype(jnp.bfloat16), wo_ref[...],
                   preferred_element_type=jnp.float32)
    iota_ul = jax.lax.broadcasted_iota(jnp.int32, (U, L), 1)
    iota_lu = jax.lax.broadcasted_iota(jnp.int32, (L, U), 0)
    deltas, ohts = [], []
    for h in range(N_HEADS):
        idx = mt_ref[0, h, :]
        onehot = (idx[:, None] == iota_ul).astype(jnp.bfloat16)
        q = q_ref[:, h * D_HEAD:(h + 1) * D_HEAD]
        k = k_ref[:, h * D_HEAD:(h + 1) * D_HEAD]
        v = v_ref[:, h * D_HEAD:(h + 1) * D_HEAD]
        q_red = jnp.dot(onehot, q, preferred_element_type=jnp.float32)
        scores = jax.lax.dot_general(q_red.astype(jnp.bfloat16), k,
                                     (((1,), (1,)), ((), ())),
                                     preferred_element_type=jnp.float32)
        scores = scores * (1.0 / math.sqrt(D_HEAD))
        mx = jnp.max(scores, axis=1, keepdims=True)
        e = jnp.exp(scores - mx)
        attn = e / jnp.sum(e, axis=1, keepdims=True)
        update = jnp.dot(attn.astype(jnp.bfloat16), v,
                         preferred_element_type=jnp.float32)
        diff = update - vmean_full[:, h * D_HEAD:(h + 1) * D_HEAD]
        wo = wo_ref[h * D_HEAD:(h + 1) * D_HEAD, :]
        deltas.append(jnp.dot(diff.astype(jnp.bfloat16), wo,
                              preferred_element_type=jnp.float32))
        ohts.append((idx[None, :] == iota_lu).astype(jnp.bfloat16))
    delta_all = jnp.concatenate(deltas, axis=0).astype(jnp.bfloat16)
    oht_all = jnp.concatenate(ohts, axis=1)
    acc = (x_ref[...] + bo_ref[...] + base
           + jnp.dot(oht_all, delta_all, preferred_element_type=jnp.float32))
    o_ref[...] = _layer_norm(acc, g_ref[...], b_ref[...])


def _ffn_body(x_ref, w1_ref, b1_ref, w2_ref, b2_ref, g_ref, bb_ref, o_ref):
    x = x_ref[...]
    h1 = jnp.dot(x.astype(jnp.bfloat16), w1_ref[...],
                 preferred_element_type=jnp.float32) + b1_ref[...]
    h1 = 0.5 * h1 * (1.0 + jax.lax.erf(h1 * (1.0 / math.sqrt(2.0))))
    y = jnp.dot(h1.astype(jnp.bfloat16), w2_ref[...],
                preferred_element_type=jnp.float32) + b2_ref[...]
    o_ref[...] = _layer_norm(x + y, g_ref[...], bb_ref[...])


def _final_body(x_ref, g_ref, b_ref, mark_ref, o_ref):
    o_ref[...] = _layer_norm(x_ref[...], g_ref[...], b_ref[...]) * mark_ref[...]


_F32 = jnp.float32


def _embed(x2, wcat, pe):
    return pl.pallas_call(
        _embed_body,
        grid=(B,),
        in_specs=[
            pl.BlockSpec((L, ENC_IN), lambda b: (b, 0)),
            pl.BlockSpec((ENC_IN, 3 * D_MODEL), lambda b: (0, 0)),
            pl.BlockSpec((L, D_MODEL), lambda b: (0, 0)),
        ],
        out_specs=pl.BlockSpec((L, D_MODEL), lambda b: (b, 0)),
        out_shape=jax.ShapeDtypeStruct((BL, D_MODEL), _F32),
    )(x2, wcat, pe)


def _qkv(x, wqkv, bqkv):
    blk = 512
    return pl.pallas_call(
        _qkv_body,
        grid=(BL // blk,),
        in_specs=[
            pl.BlockSpec((blk, D_MODEL), lambda i: (i, 0)),
            pl.BlockSpec((D_MODEL, 3 * D_MODEL), lambda i: (0, 0)),
            pl.BlockSpec((1, 3 * D_MODEL), lambda i: (0, 0)),
        ],
        out_specs=pl.BlockSpec((blk, 3 * D_MODEL), lambda i: (i, 0)),
        out_shape=jax.ShapeDtypeStruct((BL, 3 * D_MODEL), jnp.bfloat16),
    )(x, wqkv, bqkv)


_LB = 1024


def _sample(qkv, c):
    nlb = L // _LB
    return pl.pallas_call(
        _sample_body,
        grid=(nlb, B),
        in_specs=[
            pl.BlockSpec((_LB, D_MODEL), lambda lb, bb: (bb * nlb + lb, 0)),
            pl.BlockSpec((L, D_MODEL), lambda lb, bb: (bb, 1)),
            pl.BlockSpec((_LB, L), lambda lb, bb: (lb, 0)),
        ],
        out_specs=pl.BlockSpec((_LB, N_HEADS), lambda lb, bb: (bb * nlb + lb, 0)),
        out_shape=jax.ShapeDtypeStruct((BL, N_HEADS), _F32),
    )(qkv, qkv, c)


def _topk(m):
    return pl.pallas_call(
        _topk_body,
        out_shape=jax.ShapeDtypeStruct((B, N_HEADS, U), jnp.int32),
    )(m)


def _attn(mt, qkv, x, wot, bo, g, b):
    return pl.pallas_call(
        _attn_body,
        grid=(B,),
        in_specs=[
            pl.BlockSpec((1, N_HEADS, U), lambda bb: (bb, 0, 0)),
            pl.BlockSpec((L, D_MODEL), lambda bb: (bb, 0)),
            pl.BlockSpec((L, D_MODEL), lambda bb: (bb, 1)),
            pl.BlockSpec((L, D_MODEL), lambda bb: (bb, 2)),
            pl.BlockSpec((L, D_MODEL), lambda bb: (bb, 0)),
            pl.BlockSpec((D_MODEL, D_MODEL), lambda bb: (0, 0)),
            pl.BlockSpec((1, D_MODEL), lambda bb: (0, 0)),
            pl.BlockSpec((1, D_MODEL), lambda bb: (0, 0)),
            pl.BlockSpec((1, D_MODEL), lambda bb: (0, 0)),
        ],
        out_specs=pl.BlockSpec((L, D_MODEL), lambda bb: (bb, 0)),
        out_shape=jax.ShapeDtypeStruct((BL, D_MODEL), _F32),
    )(mt, qkv, qkv, qkv, x, wot, bo, g, b)


def _ffn(x, w1, b1, w2, b2, g, bb):
    blk = 512
    return pl.pallas_call(
        _ffn_body,
        grid=(BL // blk,),
        in_specs=[
            pl.BlockSpec((blk, D_MODEL), lambda i: (i, 0)),
            pl.BlockSpec((D_MODEL, D_FF), lambda i: (0, 0)),
            pl.BlockSpec((1, D_FF), lambda i: (0, 0)),
            pl.BlockSpec((D_FF, D_MODEL), lambda i: (0, 0)),
            pl.BlockSpec((1, D_MODEL), lambda i: (0, 0)),
            pl.BlockSpec((1, D_MODEL), lambda i: (0, 0)),
            pl.BlockSpec((1, D_MODEL), lambda i: (0, 0)),
        ],
        out_specs=pl.BlockSpec((blk, D_MODEL), lambda i: (i, 0)),
        out_shape=jax.ShapeDtypeStruct((BL, D_MODEL), _F32),
    )(x, w1, b1, w2, b2, g, bb)


def _final(x, g, b, mark):
    blk = 512
    return pl.pallas_call(
        _final_body,
        grid=(BL // blk,),
        in_specs=[
            pl.BlockSpec((blk, D_MODEL), lambda i: (i, 0)),
            pl.BlockSpec((1, D_MODEL), lambda i: (0, 0)),
            pl.BlockSpec((1, D_MODEL), lambda i: (0, 0)),
            pl.BlockSpec((blk, 1), lambda i: (i, 0)),
        ],
        out_specs=pl.BlockSpec((blk, D_MODEL), lambda i: (i, 0)),
        out_shape=jax.ShapeDtypeStruct((BL, D_MODEL), _F32),
    )(x, g, b, mark)


def kernel(x_enc, x_mark_enc, params):
    c = jnp.asarray(_COUNTS).astype(jnp.bfloat16)
    pe = jnp.asarray(_PE)
    w = params['token_conv_w']
    wcat = jnp.concatenate([w[:, :, 0].T, w[:, :, 1].T, w[:, :, 2].T],
                           axis=1).astype(jnp.bfloat16)
    x = _embed(x_enc.reshape(BL, ENC_IN), wcat, pe)
    for l in range(E_LAYERS):
        p = params['layer_%d' % l]
        wqkv = jnp.concatenate([p['wq'].T, p['wk'].T, p['wv'].T],
                               axis=1).astype(jnp.bfloat16)
        bqkv = jnp.concatenate([p['bq'], p['bk'], p['bv']])[None, :]
        qkv = _qkv(x, wqkv, bqkv)
        m = _sample(qkv, c)
        mt = _topk(m)
        x = _attn(mt, qkv, x, p['wo'].T.astype(jnp.bfloat16), p['bo'][None],
                  p['ln1_g'][None], p['ln1_b'][None])
        x = _ffn(x, p['conv1_w'].astype(jnp.bfloat16), p['conv1_b'][None],
                 p['conv2_w'].astype(jnp.bfloat16), p['conv2_b'][None],
                 p['ln2_g'][None], p['ln2_b'][None])
    out = _final(x, params['norm_g'][None], params['norm_b'][None],
                 x_mark_enc.reshape(BL, 1))
    return out.reshape(B, L * D_MODEL)


# speedup vs baseline: 7.0288x; 1.0240x over previous
import math

import numpy as np
import jax
import jax.numpy as jnp
from jax.experimental import pallas as pl

B, L, ENC_IN = 4, 2048, 144
D_MODEL, N_HEADS, D_FF, E_LAYERS, FACTOR = 512, 8, 2048, 2, 5
D_HEAD = D_MODEL // N_HEADS
U = min(FACTOR * int(math.ceil(math.log(L))), L)
BL = B * L
_EPS = 1e-5

_index_sample = np.random.default_rng(0).integers(0, L, size=(L, U))
_COUNTS = np.zeros((L, L), np.float32)
np.add.at(_COUNTS, (np.repeat(np.arange(L), U), _index_sample.ravel()), 1.0)


def _pos_embedding():
    pos = np.arange(L)[:, None].astype(np.float32)
    div = np.exp(np.arange(0, D_MODEL, 2).astype(np.float32)
                 * -(math.log(10000.0) / D_MODEL))
    pe = np.zeros((L, D_MODEL), dtype=np.float32)
    pe[:, 0::2] = np.sin(pos * div)
    pe[:, 1::2] = np.cos(pos * div)
    return pe


_PE = _pos_embedding()


def _layer_norm(t, g, b):
    mu = jnp.mean(t, axis=1, keepdims=True)
    var = jnp.mean((t - mu) ** 2, axis=1, keepdims=True)
    return (t - mu) / jnp.sqrt(var + _EPS) * g + b


def _embed_body(x_ref, w_ref, pe_ref, o_ref):
    x = x_ref[...].astype(jnp.bfloat16)
    w = w_ref[...]
    a0 = jnp.dot(x, w[:, :D_MODEL], preferred_element_type=jnp.float32)
    a1 = jnp.dot(x, w[:, D_MODEL:2 * D_MODEL], preferred_element_type=jnp.float32)
    a2 = jnp.dot(x, w[:, 2 * D_MODEL:], preferred_element_type=jnp.float32)
    out = (jnp.concatenate([a0[-1:], a0[:-1]], axis=0) + a1
           + jnp.concatenate([a2[1:], a2[:1]], axis=0) + pe_ref[...])
    o_ref[...] = out


def _qkv_body(x_ref, w_ref, b_ref, o_ref):
    o_ref[...] = (jnp.dot(x_ref[...].astype(jnp.bfloat16), w_ref[...],
                          preferred_element_type=jnp.float32)
                  + b_ref[...]).astype(jnp.bfloat16)


def _sample_body(q_ref, k_ref, c_ref, m_ref):
    ck = jnp.dot(c_ref[...], k_ref[...],
                 preferred_element_type=jnp.float32)
    mask = c_ref[...] > 0
    cols = []
    for h in range(N_HEADS):
        q = q_ref[:, h * D_HEAD:(h + 1) * D_HEAD]
        k = k_ref[:, h * D_HEAD:(h + 1) * D_HEAD]
        s = jax.lax.dot_general(q, k, (((1,), (1,)), ((), ())),
                                preferred_element_type=jnp.float32)
        mx = jnp.max(jnp.where(mask, s, -1e30), axis=1, keepdims=True)
        sm = jnp.sum(q.astype(jnp.float32)
                     * ck[:, h * D_HEAD:(h + 1) * D_HEAD],
                     axis=1, keepdims=True) * (1.0 / L)
        cols.append(mx - sm)
    m_ref[...] = jnp.concatenate(cols, axis=1)


def _topk_body(m_ref, o_ref):
    m = m_ref[...].reshape(B, L, N_HEADS).transpose(0, 2, 1).reshape(
        B * N_HEADS, L)
    iota = jax.lax.broadcasted_iota(jnp.int32, (B * N_HEADS, L), 1)
    cols = []
    for _ in range(U):
        mx = jnp.max(m, axis=1, keepdims=True)
        cand = jnp.where(m == mx, iota, L)
        idx = jnp.min(cand, axis=1)
        cols.append(idx[:, None])
        m = jnp.where(iota == idx[:, None], -jnp.inf, m)
    o_ref[...] = jnp.concatenate(cols, axis=1).reshape(B, N_HEADS, U)


def _attn_body(mt_ref, q_ref, k_ref, v_ref, x_ref, wo_ref, bo_ref,
               g_ref, b_ref, o_ref):
    vmean_full = jnp.mean(v_ref[...].astype(jnp.float32), axis=0,
                          keepdims=True)
    base = jnp.dot(vmean_full.astype(jnp.bfloat16), wo_ref[...],
                   preferred_element_type=jnp.float32)
    iota_ul = jax.lax.broadcasted_iota(jnp.int32, (U, L), 1)
    iota_lu = jax.lax.broadcasted_iota(jnp.int32, (L, U), 0)
    deltas, ohts = [], []
    for h in range(N_HEADS):
        idx = mt_ref[0, h, :]
        onehot = (idx[:, None] == iota_ul).astype(jnp.bfloat16)
        q = q_ref[:, h * D_HEAD:(h + 1) * D_HEAD]
        k = k_ref[:, h * D_HEAD:(h + 1) * D_HEAD]
        v = v_ref[:, h * D_HEAD:(h + 1) * D_HEAD]
        q_red = jnp.dot(onehot, q, preferred_element_type=jnp.float32)
        scores = jax.lax.dot_general(q_red.astype(jnp.bfloat16), k,
                                     (((1,), (1,)), ((), ())),
                                     preferred_element_type=jnp.float32)
        scores = scores * (1.0 / math.sqrt(D_HEAD))
        mx = jnp.max(scores, axis=1, keepdims=True)
        e = jnp.exp(scores - mx)
        attn = e / jnp.sum(e, axis=1, keepdims=True)
        update = jnp.dot(attn.astype(jnp.bfloat16), v,
                         preferred_element_type=jnp.float32)
        diff = update - vmean_full[:, h * D_HEAD:(h + 1) * D_HEAD]
        wo = wo_ref[h * D_HEAD:(h + 1) * D_HEAD, :]
        deltas.append(jnp.dot(diff.astype(jnp.bfloat16), wo,
                              preferred_element_type=jnp.float32))
        ohts.append((idx[None, :] == iota_lu).astype(jnp.bfloat16))
    delta_all = jnp.concatenate(deltas, axis=0).astype(jnp.bfloat16)
    oht_all = jnp.concatenate(ohts, axis=1)
    acc = (x_ref[...] + bo_ref[...] + base
           + jnp.dot(oht_all, delta_all, preferred_element_type=jnp.float32))
    o_ref[...] = _layer_norm(acc, g_ref[...], b_ref[...])


def _ffn_body(x_ref, w1_ref, b1_ref, w2_ref, b2_ref, g_ref, bb_ref, o_ref):
    x = x_ref[...]
    h1 = jnp.dot(x.astype(jnp.bfloat16), w1_ref[...],
                 preferred_element_type=jnp.float32) + b1_ref[...]
    h1 = 0.5 * h1 * (1.0 + jax.lax.erf(h1 * (1.0 / math.sqrt(2.0))))
    y = jnp.dot(h1.astype(jnp.bfloat16), w2_ref[...],
                preferred_element_type=jnp.float32) + b2_ref[...]
    o_ref[...] = _layer_norm(x + y, g_ref[...], bb_ref[...])


def _final_body(x_ref, g_ref, b_ref, mark_ref, o_ref):
    o_ref[...] = _layer_norm(x_ref[...], g_ref[...], b_ref[...]) * mark_ref[...]


def _ffn_final_body(x_ref, w1_ref, b1_ref, w2_ref, b2_ref, g_ref, bb_ref,
                    g2_ref, b2f_ref, mark_ref, o_ref):
    x = x_ref[...]
    h1 = jnp.dot(x.astype(jnp.bfloat16), w1_ref[...],
                 preferred_element_type=jnp.float32) + b1_ref[...]
    h1 = 0.5 * h1 * (1.0 + jax.lax.erf(h1 * (1.0 / math.sqrt(2.0))))
    y = jnp.dot(h1.astype(jnp.bfloat16), w2_ref[...],
                preferred_element_type=jnp.float32) + b2_ref[...]
    t = _layer_norm(x + y, g_ref[...], bb_ref[...])
    o_ref[...] = _layer_norm(t, g2_ref[...], b2f_ref[...]) * mark_ref[...]


_F32 = jnp.float32


def _embed(x2, wcat, pe):
    return pl.pallas_call(
        _embed_body,
        grid=(B,),
        in_specs=[
            pl.BlockSpec((L, ENC_IN), lambda b: (b, 0)),
            pl.BlockSpec((ENC_IN, 3 * D_MODEL), lambda b: (0, 0)),
            pl.BlockSpec((L, D_MODEL), lambda b: (0, 0)),
        ],
        out_specs=pl.BlockSpec((L, D_MODEL), lambda b: (b, 0)),
        out_shape=jax.ShapeDtypeStruct((BL, D_MODEL), _F32),
    )(x2, wcat, pe)


def _qkv(x, wqkv, bqkv):
    blk = 512
    return pl.pallas_call(
        _qkv_body,
        grid=(BL // blk,),
        in_specs=[
            pl.BlockSpec((blk, D_MODEL), lambda i: (i, 0)),
            pl.BlockSpec((D_MODEL, 3 * D_MODEL), lambda i: (0, 0)),
            pl.BlockSpec((1, 3 * D_MODEL), lambda i: (0, 0)),
        ],
        out_specs=pl.BlockSpec((blk, 3 * D_MODEL), lambda i: (i, 0)),
        out_shape=jax.ShapeDtypeStruct((BL, 3 * D_MODEL), jnp.bfloat16),
    )(x, wqkv, bqkv)


_LB = 1024


def _sample(qkv, c):
    nlb = L // _LB
    return pl.pallas_call(
        _sample_body,
        grid=(nlb, B),
        in_specs=[
            pl.BlockSpec((_LB, D_MODEL), lambda lb, bb: (bb * nlb + lb, 0)),
            pl.BlockSpec((L, D_MODEL), lambda lb, bb: (bb, 1)),
            pl.BlockSpec((_LB, L), lambda lb, bb: (lb, 0)),
        ],
        out_specs=pl.BlockSpec((_LB, N_HEADS), lambda lb, bb: (bb * nlb + lb, 0)),
        out_shape=jax.ShapeDtypeStruct((BL, N_HEADS), _F32),
    )(qkv, qkv, c)


def _topk(m):
    return pl.pallas_call(
        _topk_body,
        out_shape=jax.ShapeDtypeStruct((B, N_HEADS, U), jnp.int32),
    )(m)


def _attn(mt, qkv, x, wot, bo, g, b):
    return pl.pallas_call(
        _attn_body,
        grid=(B,),
        in_specs=[
            pl.BlockSpec((1, N_HEADS, U), lambda bb: (bb, 0, 0)),
            pl.BlockSpec((L, D_MODEL), lambda bb: (bb, 0)),
            pl.BlockSpec((L, D_MODEL), lambda bb: (bb, 1)),
            pl.BlockSpec((L, D_MODEL), lambda bb: (bb, 2)),
            pl.BlockSpec((L, D_MODEL), lambda bb: (bb, 0)),
            pl.BlockSpec((D_MODEL, D_MODEL), lambda bb: (0, 0)),
            pl.BlockSpec((1, D_MODEL), lambda bb: (0, 0)),
            pl.BlockSpec((1, D_MODEL), lambda bb: (0, 0)),
            pl.BlockSpec((1, D_MODEL), lambda bb: (0, 0)),
        ],
        out_specs=pl.BlockSpec((L, D_MODEL), lambda bb: (bb, 0)),
        out_shape=jax.ShapeDtypeStruct((BL, D_MODEL), _F32),
    )(mt, qkv, qkv, qkv, x, wot, bo, g, b)


def _ffn(x, w1, b1, w2, b2, g, bb):
    blk = 512
    return pl.pallas_call(
        _ffn_body,
        grid=(BL // blk,),
        in_specs=[
            pl.BlockSpec((blk, D_MODEL), lambda i: (i, 0)),
            pl.BlockSpec((D_MODEL, D_FF), lambda i: (0, 0)),
            pl.BlockSpec((1, D_FF), lambda i: (0, 0)),
            pl.BlockSpec((D_FF, D_MODEL), lambda i: (0, 0)),
            pl.BlockSpec((1, D_MODEL), lambda i: (0, 0)),
            pl.BlockSpec((1, D_MODEL), lambda i: (0, 0)),
            pl.BlockSpec((1, D_MODEL), lambda i: (0, 0)),
        ],
        out_specs=pl.BlockSpec((blk, D_MODEL), lambda i: (i, 0)),
        out_shape=jax.ShapeDtypeStruct((BL, D_MODEL), _F32),
    )(x, w1, b1, w2, b2, g, bb)


def _ffn_final(x, w1, b1, w2, b2, g, bb, g2, b2f, mark):
    blk = 512
    return pl.pallas_call(
        _ffn_final_body,
        grid=(BL // blk,),
        in_specs=[
            pl.BlockSpec((blk, D_MODEL), lambda i: (i, 0)),
            pl.BlockSpec((D_MODEL, D_FF), lambda i: (0, 0)),
            pl.BlockSpec((1, D_FF), lambda i: (0, 0)),
            pl.BlockSpec((D_FF, D_MODEL), lambda i: (0, 0)),
            pl.BlockSpec((1, D_MODEL), lambda i: (0, 0)),
            pl.BlockSpec((1, D_MODEL), lambda i: (0, 0)),
            pl.BlockSpec((1, D_MODEL), lambda i: (0, 0)),
            pl.BlockSpec((1, D_MODEL), lambda i: (0, 0)),
            pl.BlockSpec((1, D_MODEL), lambda i: (0, 0)),
            pl.BlockSpec((blk, 1), lambda i: (i, 0)),
        ],
        out_specs=pl.BlockSpec((blk, D_MODEL), lambda i: (i, 0)),
        out_shape=jax.ShapeDtypeStruct((BL, D_MODEL), _F32),
    )(x, w1, b1, w2, b2, g, bb, g2, b2f, mark)


def _final(x, g, b, mark):
    blk = 512
    return pl.pallas_call(
        _final_body,
        grid=(BL // blk,),
        in_specs=[
            pl.BlockSpec((blk, D_MODEL), lambda i: (i, 0)),
            pl.BlockSpec((1, D_MODEL), lambda i: (0, 0)),
            pl.BlockSpec((1, D_MODEL), lambda i: (0, 0)),
            pl.BlockSpec((blk, 1), lambda i: (i, 0)),
        ],
        out_specs=pl.BlockSpec((blk, D_MODEL), lambda i: (i, 0)),
        out_shape=jax.ShapeDtypeStruct((BL, D_MODEL), _F32),
    )(x, g, b, mark)


def kernel(x_enc, x_mark_enc, params):
    c = jnp.asarray(_COUNTS).astype(jnp.bfloat16)
    pe = jnp.asarray(_PE)
    w = params['token_conv_w']
    wcat = jnp.concatenate([w[:, :, 0].T, w[:, :, 1].T, w[:, :, 2].T],
                           axis=1).astype(jnp.bfloat16)
    x = _embed(x_enc.reshape(BL, ENC_IN), wcat, pe)
    for l in range(E_LAYERS):
        p = params['layer_%d' % l]
        wqkv = jnp.concatenate([p['wq'].T, p['wk'].T, p['wv'].T],
                               axis=1).astype(jnp.bfloat16)
        bqkv = jnp.concatenate([p['bq'], p['bk'], p['bv']])[None, :]
        qkv = _qkv(x, wqkv, bqkv)
        m = _sample(qkv, c)
        mt = _topk(m)
        x = _attn(mt, qkv, x, p['wo'].T.astype(jnp.bfloat16), p['bo'][None],
                  p['ln1_g'][None], p['ln1_b'][None])
        if l < E_LAYERS - 1:
            x = _ffn(x, p['conv1_w'].astype(jnp.bfloat16), p['conv1_b'][None],
                     p['conv2_w'].astype(jnp.bfloat16), p['conv2_b'][None],
                     p['ln2_g'][None], p['ln2_b'][None])
        else:
            x = _ffn_final(x, p['conv1_w'].astype(jnp.bfloat16),
                           p['conv1_b'][None],
                           p['conv2_w'].astype(jnp.bfloat16),
                           p['conv2_b'][None],
                           p['ln2_g'][None], p['ln2_b'][None],
                           params['norm_g'][None], params['norm_b'][None],
                           x_mark_enc.reshape(BL, 1))
    return x.reshape(B, L * D_MODEL)
